# baseline (speedup 1.0000x reference)
import functools

import jax
import jax.numpy as jnp
from jax import lax
from jax.experimental import pallas as pl
from jax.experimental.pallas import tpu as pltpu
from jax.experimental.pallas import tpu_sc as plsc

VOCAB_P1 = 1000001
VOCAB_PAD = 1000064
NTILE = VOCAB_PAD // 128
EMBED_DIM = 32
BATCH = 16384
_LANES = 128

_info = plsc.get_sparse_core_info()
_NC, _NS = _info.num_cores, _info.num_subcores
_NW = _NC * _NS

_B1 = 12288
_B2 = BATCH - _B1
_B1_PER_W = _B1 // _NW
_B2_PER_W = _B2 // _NW
_WAVE = 16


def _slab_body(idx_hbm, tab_hbm, out_hbm, idx_v, slab_v, out_v, sem):
    wid = lax.axis_index("s") * _NC + lax.axis_index("c")
    base = wid * _B1_PER_W
    pltpu.sync_copy(idx_hbm.at[pl.ds(base, _B1_PER_W)], idx_v)
    c_lo = lax.iota(jnp.int32, 16)
    c_hi = c_lo + 16

    def wave(g, carry):
        vec = idx_v[pl.ds(g * _WAVE, _WAVE)]
        copies = []
        for k in range(_WAVE):
            blk = pl.multiple_of(vec[k] & ~(_LANES - 1), _LANES)
            copies.append(
                pltpu.async_copy(
                    tab_hbm.at[:, pl.ds(blk, _LANES)], slab_v.at[k], sem
                )
            )
        lane = vec & (_LANES - 1)
        for k in range(_WAVE):
            copies[k].wait()
            l_vec = jnp.full((16,), lane[k], dtype=jnp.int32)
            lo = plsc.load_gather(slab_v.at[k], [c_lo, l_vec])
            hi = plsc.load_gather(slab_v.at[k], [c_hi, l_vec])
            j_vec = jnp.full((16,), g * _WAVE + k, dtype=jnp.int32)
            plsc.store_scatter(out_v, [c_lo, j_vec], lo)
            plsc.store_scatter(out_v, [c_hi, j_vec], hi)
        return carry

    lax.fori_loop(0, _B1_PER_W // _WAVE, wave, None)
    pltpu.sync_copy(out_v, out_hbm.at[:, pl.ds(base, _B1_PER_W)])


def _flat_body(idx_hbm, tabf_hbm, out_hbm, idx_v, flat_v, rows_v, out_v, sem):
    wid = lax.axis_index("s") * _NC + lax.axis_index("c")
    base = wid * _B2_PER_W
    pltpu.sync_copy(idx_hbm.at[pl.ds(_B1 + base, _B2_PER_W)], idx_v)

    def build(i, carry):
        r = idx_v[pl.ds(i * 16, 16)]
        hi = (r >> 7) * 1024 + (r & 127)
        for c in range(EMBED_DIM):
            off = (c >> 3) * (NTILE * 1024) + (c & 7) * 128
            flat_v[pl.ds(c * _B2_PER_W + i * 16, 16)] = hi + off
        return carry

    lax.fori_loop(0, _B2_PER_W // 16, build, None)

    copies = []
    for c in range(EMBED_DIM):
        copies.append(
            pltpu.async_copy(
                tabf_hbm.at[flat_v.at[pl.ds(c * _B2_PER_W, _B2_PER_W)]],
                rows_v.at[pl.ds(c * _B2_PER_W, _B2_PER_W)],
                sem,
            )
        )
    for cp in copies:
        cp.wait()

    c_vec = lax.iota(jnp.int32, 16)
    n_chunks = EMBED_DIM * _B2_PER_W // 16
    per_c = _B2_PER_W // 16

    def regroup(k, carry):
        v = rows_v[pl.ds(k * 16, 16)]
        c_idx = jnp.full((16,), k // per_c, dtype=jnp.int32)
        j_vec = jnp.full((16,), (k % per_c) * 16, dtype=jnp.int32) + c_vec
        plsc.store_scatter(out_v, [c_idx, j_vec], v)
        return carry

    lax.fori_loop(0, n_chunks, regroup, None)
    pltpu.sync_copy(out_v, out_hbm.at[:, pl.ds(base, _B2_PER_W)])


@jax.jit
def kernel(item_id, id_table):
    item_id = item_id.astype(jnp.int32)
    mesh = plsc.VectorSubcoreMesh(core_axis_name="c", subcore_axis_name="s")

    slab_gather = functools.partial(
        pl.kernel,
        mesh=mesh,
        out_type=jax.ShapeDtypeStruct((EMBED_DIM, _B1), jnp.float32),
        scratch_types=[
            pltpu.VMEM((_B1_PER_W,), jnp.int32),
            pltpu.VMEM((_WAVE, EMBED_DIM, _LANES), jnp.float32),
            pltpu.VMEM((EMBED_DIM, _B1_PER_W), jnp.float32),
            pltpu.SemaphoreType.DMA,
        ],
        compiler_params=pltpu.CompilerParams(needs_layout_passes=False),
    )(_slab_body)

    flat_gather = functools.partial(
        pl.kernel,
        mesh=mesh,
        out_type=jax.ShapeDtypeStruct((EMBED_DIM, _B2), jnp.float32),
        scratch_types=[
            pltpu.VMEM((_B2_PER_W,), jnp.int32),
            pltpu.VMEM((EMBED_DIM * _B2_PER_W,), jnp.int32),
            pltpu.VMEM((EMBED_DIM * _B2_PER_W,), jnp.float32),
            pltpu.VMEM((EMBED_DIM, _B2_PER_W), jnp.float32),
            pltpu.SemaphoreType.DMA,
        ],
        compiler_params=pltpu.CompilerParams(
            use_tc_tiling_on_sc=False, needs_layout_passes=False
        ),
    )(_flat_body)

    out1 = slab_gather(item_id, id_table.T)
    tab_pad = jnp.pad(id_table, ((0, VOCAB_PAD - VOCAB_P1), (0, 0)))
    tab_flat = (
        tab_pad.T.reshape(4, 8, NTILE, 128).transpose(0, 2, 1, 3).reshape(-1)
    )
    out2 = flat_gather(item_id, tab_flat)
    out_t = jnp.concatenate([out1, out2], axis=1)
    return out_t.T


# speedup vs baseline: 1.5933x; 1.5933x over previous
import functools

import jax
import jax.numpy as jnp
from jax import lax
from jax.experimental import pallas as pl
from jax.experimental.pallas import tpu as pltpu
from jax.experimental.pallas import tpu_sc as plsc

VOCAB_P1 = 1000001
VOCAB_PAD = 1000064
NTILE = VOCAB_PAD // 128
EMBED_DIM = 32
BATCH = 16384

_info = plsc.get_sparse_core_info()
_NC, _NS = _info.num_cores, _info.num_subcores
_NW = _NC * _NS
_B_PER_W = BATCH // _NW


def _gather_body(idx_hbm, tabf_hbm, out_hbm, idx_v, flat_v, rows_v, out_v, sem):
    wid = lax.axis_index("s") * _NC + lax.axis_index("c")
    base = wid * _B_PER_W
    pltpu.sync_copy(idx_hbm.at[pl.ds(base, _B_PER_W)], idx_v)

    def build(i, carry):
        r = idx_v[pl.ds(i * 16, 16)]
        hi = (r >> 7) * 1024 + (r & 127)
        for c in range(EMBED_DIM):
            off = (c >> 3) * (NTILE * 1024) + (c & 7) * 128
            flat_v[pl.ds(c * _B_PER_W + i * 16, 16)] = hi + off
        return carry

    lax.fori_loop(0, _B_PER_W // 16, build, None)

    copies = []
    for c in range(EMBED_DIM):
        copies.append(
            pltpu.async_copy(
                tabf_hbm.at[flat_v.at[pl.ds(c * _B_PER_W, _B_PER_W)]],
                rows_v.at[pl.ds(c * _B_PER_W, _B_PER_W)],
                sem,
            )
        )
    for cp in copies:
        cp.wait()

    c_vec = lax.iota(jnp.int32, 16)

    def regroup(i, carry):
        for c2 in range(2):
            v = rows_v[pl.ds((i * 2 + c2) * 16, 16)]
            cc = i * 2 + c2
            c_idx = jnp.full((16,), cc // 32, dtype=jnp.int32)
            j_vec = jnp.full((16,), (cc % 32) * 16, dtype=jnp.int32) + c_vec
            plsc.store_scatter(out_v, [c_idx, j_vec], v)
        return carry

    lax.fori_loop(0, EMBED_DIM * _B_PER_W // 32, regroup, None)
    pltpu.sync_copy(out_v, out_hbm.at[:, pl.ds(base, _B_PER_W)])


@jax.jit
def kernel(item_id, id_table):
    tab_pad = jnp.pad(id_table, ((0, VOCAB_PAD - VOCAB_P1), (0, 0)))
    v2 = tab_pad.T.reshape(4, 8, NTILE, 128)
    v = v2.transpose(0, 2, 1, 3)
    tab_flat = v.reshape(-1)
    mesh = plsc.VectorSubcoreMesh(core_axis_name="c", subcore_axis_name="s")
    gather = functools.partial(
        pl.kernel,
        mesh=mesh,
        out_type=jax.ShapeDtypeStruct((EMBED_DIM, BATCH), jnp.float32),
        scratch_types=[
            pltpu.VMEM((_B_PER_W,), jnp.int32),
            pltpu.VMEM((EMBED_DIM * _B_PER_W,), jnp.int32),
            pltpu.VMEM((EMBED_DIM * _B_PER_W,), jnp.float32),
            pltpu.VMEM((EMBED_DIM, _B_PER_W), jnp.float32),
            pltpu.SemaphoreType.DMA,
        ],
        compiler_params=pltpu.CompilerParams(
            use_tc_tiling_on_sc=False, needs_layout_passes=False
        ),
    )(_gather_body)
    out_t = gather(item_id.astype(jnp.int32), tab_flat)
    return out_t.T
